# Initial kernel scaffold; baseline (speedup 1.0000x reference)
import jax
import jax.numpy as jnp
from jax.experimental import pallas as pl


def kernel(x, edge_index, edge_attr, batch, solvent_fingerprint, params):
    raise NotImplementedError("write your pallas kernel here")


# speedup vs baseline: 5.6114x; 5.6114x over previous
import functools

import jax
import jax.numpy as jnp
from jax import lax
from jax.experimental import pallas as pl
from jax.experimental.pallas import tpu as pltpu
from jax.experimental.pallas import tpu_sc as plsc

_N = 10000
_D = 128
_HID = 32
_G = 128
_NCORES = 2
_NSUB = 16
_NTILES = _NCORES * _NSUB
_CHUNK = 128
_K = 81
_EPAD = _NTILES * _K * _CHUNK
_NP = 10112
_WB = _NP // _NSUB
_YSTAGE = _N - (_NSUB - 1) * _WB


def _zero_acc_slice(zbuf, acc, base):
    for k in range(_WB // _CHUNK):
        pltpu.sync_copy(zbuf, acc.at[pl.ds(base + k * _CHUNK, _CHUNK)])
    rem = _WB - (_WB // _CHUNK) * _CHUNK
    pltpu.sync_copy(zbuf.at[pl.ds(0, rem)],
                    acc.at[pl.ds(base + _WB - rem, rem)])


def _zero_vmem(buf, width):
    zv = jnp.zeros((16,), jnp.float32)

    def zrow(i, carry):
        for off in range(0, width, 16):
            buf[i, pl.ds(off, 16)] = zv
        return carry

    lax.fori_loop(0, _CHUNK, zrow, 0)


_HD = _D // 2


def _sc_scatter64(x, src_r, dst_r):
    mesh = plsc.VectorSubcoreMesh(core_axis_name="c", subcore_axis_name="s")

    @functools.partial(
        pl.kernel,
        mesh=mesh,
        compiler_params=pltpu.CompilerParams(use_tc_tiling_on_sc=False),
        out_type=jax.ShapeDtypeStruct((_NCORES, _NP, _HD), jnp.float32),
        scratch_types=[
            pltpu.VMEM((_K, _CHUNK), jnp.int32),
            pltpu.VMEM((_K, _CHUNK), jnp.int32),
            pltpu.VMEM((_CHUNK, _HD), jnp.float32),
            pltpu.VMEM((_CHUNK, _HD), jnp.float32),
            pltpu.VMEM_SHARED((_NP, _HD), jnp.float32),
            pltpu.SemaphoreType.DMA,
        ],
    )
    def scat(x_hbm, src_hbm, dst_hbm, out_hbm,
             src_v, dst_v, rows_v, zbuf, acc, sem):
        c = lax.axis_index("c")
        s = lax.axis_index("s")
        wid = c * _NSUB + s
        base = s * _WB

        _zero_vmem(zbuf, _HD)
        _zero_acc_slice(zbuf, acc, base)
        pltpu.sync_copy(src_hbm.at[wid], src_v)
        pltpu.sync_copy(dst_hbm.at[wid], dst_v)
        plsc.subcore_barrier()

        def chunk(j, carry):
            pltpu.async_copy(x_hbm.at[src_v.at[j]], rows_v, sem).wait()
            pltpu.sync_copy(rows_v, acc.at[dst_v.at[j]], add=True)
            return carry

        lax.fori_loop(0, _K, chunk, 0)
        plsc.subcore_barrier()

        for k in range(_WB // _CHUNK):
            pltpu.sync_copy(acc.at[pl.ds(base + k * _CHUNK, _CHUNK)], zbuf)
            pltpu.sync_copy(zbuf, out_hbm.at[c, pl.ds(base + k * _CHUNK,
                                                      _CHUNK)])
        rem = _WB - (_WB // _CHUNK) * _CHUNK
        pltpu.sync_copy(acc.at[pl.ds(base + _WB - rem, rem)],
                        zbuf.at[pl.ds(0, rem)])
        pltpu.sync_copy(zbuf.at[pl.ds(0, rem)],
                        out_hbm.at[c, pl.ds(base + _WB - rem, rem)])

    return scat(x, src_r, dst_r)


def _sc_scatter32(x, src_r, dst_r):
    mesh = plsc.VectorSubcoreMesh(core_axis_name="c", subcore_axis_name="s")

    @functools.partial(
        pl.kernel,
        mesh=mesh,
        compiler_params=pltpu.CompilerParams(use_tc_tiling_on_sc=False),
        out_type=jax.ShapeDtypeStruct((_NCORES, _NP, _HID), jnp.float32),
        scratch_types=[
            pltpu.VMEM((_K, _CHUNK), jnp.int32),
            pltpu.VMEM((_K, _CHUNK), jnp.int32),
            pltpu.VMEM((_CHUNK, _HID), jnp.float32),
            pltpu.VMEM((_CHUNK, _HID), jnp.float32),
            pltpu.VMEM((_WB, _HID), jnp.float32),
            pltpu.VMEM_SHARED((_N, _HID), jnp.float32),
            pltpu.VMEM_SHARED((_NP, _HID), jnp.float32),
            pltpu.SemaphoreType.DMA,
        ],
    )
    def scat(x_hbm, src_hbm, dst_hbm, out_hbm,
             src_v, dst_v, rows_v, zbuf, stage_v, x_sh, acc, sem):
        c = lax.axis_index("c")
        s = lax.axis_index("s")
        wid = c * _NSUB + s
        base = s * _WB

        _zero_vmem(zbuf, _HID)
        _zero_acc_slice(zbuf, acc, base)

        @pl.when(s < _NSUB - 1)
        def _():
            pltpu.sync_copy(x_hbm.at[pl.ds(base, _WB)], stage_v)
            pltpu.sync_copy(stage_v, x_sh.at[pl.ds(base, _WB)])

        @pl.when(s == _NSUB - 1)
        def _():
            pltpu.sync_copy(x_hbm.at[pl.ds((_NSUB - 1) * _WB, _YSTAGE)],
                            stage_v.at[pl.ds(0, _YSTAGE)])
            pltpu.sync_copy(stage_v.at[pl.ds(0, _YSTAGE)],
                            x_sh.at[pl.ds((_NSUB - 1) * _WB, _YSTAGE)])

        pltpu.sync_copy(src_hbm.at[wid], src_v)
        pltpu.sync_copy(dst_hbm.at[wid], dst_v)
        plsc.subcore_barrier()

        def chunk(j, carry):
            pltpu.async_copy(x_sh.at[src_v.at[j]], rows_v, sem).wait()
            pltpu.sync_copy(rows_v, acc.at[dst_v.at[j]], add=True)
            return carry

        lax.fori_loop(0, _K, chunk, 0)
        plsc.subcore_barrier()

        pltpu.sync_copy(acc.at[pl.ds(base, _WB)], stage_v)
        pltpu.sync_copy(stage_v, out_hbm.at[c, pl.ds(base, _WB)])

    return scat(x, src_r, dst_r)


def _gin_bn_block(x_ref, p_ref, wa_ref, ba_ref, wb_ref, bb_ref, g_ref,
                  be_ref):
    agg = p_ref[0][:_N] + p_ref[1][:_N]
    h0 = x_ref[...] + agg
    u = jnp.maximum(jnp.dot(h0, wa_ref[...],
                            preferred_element_type=jnp.float32)
                    + ba_ref[...], 0.0)
    h = jnp.dot(u, wb_ref[...],
                preferred_element_type=jnp.float32) + bb_ref[...]
    m = jnp.mean(h, axis=0, keepdims=True)
    v = jnp.mean((h - m) ** 2, axis=0, keepdims=True)
    return jnp.maximum((h - m) / jnp.sqrt(v + 1e-5) * g_ref[...] + be_ref[...],
                       0.0)


def _mid_body(x_ref, p_ref, wa_ref, ba_ref, wb_ref, bb_ref, g_ref, be_ref,
              o_ref):
    o_ref[...] = _gin_bn_block(x_ref, p_ref, wa_ref, ba_ref, wb_ref, bb_ref,
                               g_ref, be_ref)


def _first_body(x_ref, pa_ref, pb_ref, wa_ref, ba_ref, wb_ref, bb_ref,
                g_ref, be_ref, o_ref):
    agg = jnp.concatenate(
        [pa_ref[0][:_N] + pa_ref[1][:_N], pb_ref[0][:_N] + pb_ref[1][:_N]],
        axis=1)
    h0 = x_ref[...] + agg
    u = jnp.maximum(jnp.dot(h0, wa_ref[...],
                            preferred_element_type=jnp.float32)
                    + ba_ref[...], 0.0)
    h = jnp.dot(u, wb_ref[...],
                preferred_element_type=jnp.float32) + bb_ref[...]
    m = jnp.mean(h, axis=0, keepdims=True)
    v = jnp.mean((h - m) ** 2, axis=0, keepdims=True)
    o_ref[...] = jnp.maximum(
        (h - m) / jnp.sqrt(v + 1e-5) * g_ref[...] + be_ref[...], 0.0)


def _last_body(x_ref, p_ref, wa_ref, ba_ref, wb_ref, bb_ref, g_ref, be_ref,
               batch_ref, solv_ref, wg_ref, bg_ref, ws1_ref, bs1_ref,
               ws2_ref, bs2_ref, wf1a_ref, wf1b_ref, bf1_ref, wf2_ref,
               bf2_ref, wo_ref, bo_ref, o_ref):
    xn = _gin_bn_block(x_ref, p_ref, wa_ref, ba_ref, wb_ref, bb_ref, g_ref,
                       be_ref)
    ids = batch_ref[...]
    seg = lax.broadcasted_iota(jnp.int32, (_N, _G), 1)
    onehot = (ids == seg).astype(jnp.float32)
    pooled = lax.dot_general(onehot, xn, (((0,), (0,)), ((), ())),
                             preferred_element_type=jnp.float32,
                             precision=lax.Precision.HIGHEST)
    hg = jnp.maximum(jnp.dot(pooled, wg_ref[...],
                             preferred_element_type=jnp.float32)
                     + bg_ref[...], 0.0)
    s1 = jnp.maximum(jnp.dot(solv_ref[...], ws1_ref[...],
                             preferred_element_type=jnp.float32)
                     + bs1_ref[...], 0.0)
    s2 = jnp.maximum(jnp.dot(s1, ws2_ref[...],
                             preferred_element_type=jnp.float32)
                     + bs2_ref[...], 0.0)
    z1 = jnp.maximum(jnp.dot(hg, wf1a_ref[...],
                             preferred_element_type=jnp.float32)
                     + jnp.dot(s2, wf1b_ref[...],
                               preferred_element_type=jnp.float32)
                     + bf1_ref[...], 0.0)
    z2 = jnp.maximum(jnp.dot(z1, wf2_ref[...],
                             preferred_element_type=jnp.float32)
                     + bf2_ref[...], 0.0)
    o_ref[...] = (jnp.dot(z2, wo_ref[...], preferred_element_type=jnp.float32)
                  + bo_ref[...])


def kernel(x, edge_index, edge_attr, batch, solvent_fingerprint, params):
    p = params
    src = edge_index[0]
    dst = edge_index[1]
    padn = _EPAD - src.shape[0]
    src_r = jnp.concatenate(
        [src, jnp.zeros((padn,), jnp.int32)]).reshape(_NTILES, _K, _CHUNK)
    dst_r = jnp.concatenate(
        [dst, jnp.full((padn,), _N, jnp.int32)]).reshape(_NTILES, _K, _CHUNK)
    batch2 = batch.reshape(_N, 1)

    def r1(a):
        return a.reshape(1, -1)

    part_a = _sc_scatter64(x[:, :_HD], src_r, dst_r)
    part_b = _sc_scatter64(x[:, _HD:], src_r, dst_r)
    h = pl.pallas_call(
        _first_body,
        out_shape=jax.ShapeDtypeStruct((_N, _HID), jnp.float32),
    )(x, part_a, part_b, p["W1a"], r1(p["b1a"]), p["W1b"], r1(p["b1b"]),
      r1(p["g1"]), r1(p["be1"]))

    for i in range(2, 5):
        part = _sc_scatter32(h, src_r, dst_r)
        h = pl.pallas_call(
            _mid_body,
            out_shape=jax.ShapeDtypeStruct((_N, _HID), jnp.float32),
        )(h, part, p[f"W{i}a"], r1(p[f"b{i}a"]), p[f"W{i}b"], r1(p[f"b{i}b"]),
          r1(p[f"g{i}"]), r1(p[f"be{i}"]))

    part = _sc_scatter32(h, src_r, dst_r)
    out = pl.pallas_call(
        _last_body,
        out_shape=jax.ShapeDtypeStruct((_G, 1), jnp.float32),
    )(h, part, p["W5a"], r1(p["b5a"]), p["W5b"], r1(p["b5b"]), r1(p["g5"]),
      r1(p["be5"]), batch2, solvent_fingerprint,
      p["Wg"], r1(p["bg"]), p["Ws1"], r1(p["bs1"]), p["Ws2"], r1(p["bs2"]),
      p["Wf1"][:_G], p["Wf1"][_G:], r1(p["bf1"]), p["Wf2"], r1(p["bf2"]),
      p["Wo"], r1(p["bo"]))
    return out
